# baseline (speedup 1.0000x reference)
import jax
import jax.numpy as jnp
from jax import lax
from jax.experimental import pallas as pl
from jax.experimental.pallas import tpu as pltpu

D_MODEL = 64
D_STATE = 16
D_INNER = 128
DT_RANK = 4
K = 4
H = 64
W = 64
L = H * W

T_SCAN = 256
NCH = L // T_SCAN
T_POST = 1024
BH = T_POST // W

_HI = jax.lax.Precision.HIGHEST


def _silu(x):
    return x * (1.0 / (1.0 + jnp.exp(-x)))


def _softplus(x):
    return jnp.maximum(x, 0.0) + jnp.log(1.0 + jnp.exp(-jnp.abs(x)))


def _dotT(a, b):
    return lax.dot_general(a, b, (((1,), (1,)), ((), ())),
                           precision=_HI, preferred_element_type=jnp.float32)


def _pre_kernel(x_ref, win_ref, cw_ref, cb_ref, xf_ref, xt_ref, z_ref):
    x = x_ref[...].reshape(L, D_MODEL)
    w = win_ref[0]
    xz = _dotT(x, w)
    z_ref[...] = xz[:, D_INNER:].reshape(1, L, D_INNER)
    xi = xz[:, :D_INNER].reshape(H, W, D_INNER)

    cw = cw_ref[0]
    zW = jnp.zeros((H, 1, D_INNER), jnp.float32)
    zH = jnp.zeros((1, W, D_INNER), jnp.float32)
    xm = jnp.concatenate([xi[:, 1:, :], zW], axis=1)
    xp = jnp.concatenate([zW, xi[:, :-1, :]], axis=1)
    acc = jnp.zeros((H, W, D_INNER), jnp.float32)
    for kh in range(3):
        for kw in range(3):
            src = (xp, xi, xm)[kw]
            dy = kh - 1
            if dy == -1:
                sh = jnp.concatenate([zH, src[:-1]], axis=0)
            elif dy == 0:
                sh = src
            else:
                sh = jnp.concatenate([src[1:], zH], axis=0)
            acc = acc + sh * cw[kh * 3 + kw][None, None, :]
    acc = acc + cb_ref[0][None, :, :]
    xc = _silu(acc)
    xf_ref[...] = xc.reshape(1, L, D_INNER)
    xt_ref[...] = jnp.swapaxes(xc, 0, 1).reshape(1, L, D_INNER)


def _pre_call(xstk, win, cw, cb):
    return pl.pallas_call(
        _pre_kernel,
        grid=(2,),
        in_specs=[
            pl.BlockSpec((1, H, W, D_MODEL), lambda c: (c, 0, 0, 0)),
            pl.BlockSpec((1, 2 * D_INNER, D_MODEL), lambda c: (c, 0, 0)),
            pl.BlockSpec((1, 9, D_INNER), lambda c: (c, 0, 0)),
            pl.BlockSpec((1, 1, D_INNER), lambda c: (c, 0, 0)),
        ],
        out_specs=[
            pl.BlockSpec((1, L, D_INNER), lambda c: (c, 0, 0)),
            pl.BlockSpec((1, L, D_INNER), lambda c: (c, 0, 0)),
            pl.BlockSpec((1, L, D_INNER), lambda c: (c, 0, 0)),
        ],
        out_shape=[
            jax.ShapeDtypeStruct((2, L, D_INNER), jnp.float32),
            jax.ShapeDtypeStruct((2, L, D_INNER), jnp.float32),
            jax.ShapeDtypeStruct((2, L, D_INNER), jnp.float32),
        ],
        compiler_params=pltpu.CompilerParams(
            dimension_semantics=("arbitrary",),
            vmem_limit_bytes=48 * 1024 * 1024,
        ),
        name="ss2d_pre",
    )(xstk, win, cw, cb)


def _scan_kernel(xf_ref, xt_ref, xpw_ref, dtwc_ref, b0_ref, b1_ref,
                 a0_ref, a1_ref, d0_ref, d1_ref,
                 yf_ref, yt_ref,
                 dAA_ref, dBA_ref, dAB_ref, dBB_ref,
                 hA_ref, hB_ref):
    cc = pl.program_id(0)
    i = pl.program_id(1)
    T = T_SCAN

    @pl.when(i == 0)
    def _():
        hA_ref[...] = jnp.zeros_like(hA_ref)
        hB_ref[...] = jnp.zeros_like(hB_ref)

    cb_list = []
    u_list = []
    for j in range(2):
        jsl = slice(j * 128, (j + 1) * 128)
        u_ref = (xf_ref, xt_ref)[j]
        u0 = u_ref[0]
        u1 = u_ref[1]
        u_list.append((u0, u1))
        xd = _dotT(u0, xpw_ref[j])
        r = xd[:, :DT_RANK]
        Bm = xd[:, DT_RANK:DT_RANK + D_STATE]
        Cm = xd[:, DT_RANK + D_STATE:DT_RANK + 2 * D_STATE]
        dd = _dotT(r, dtwc_ref[j])
        d1 = _softplus(dd[:, :D_INNER] + b1_ref[j])
        d0 = _softplus(dd[:, D_INNER:] + b0_ref[j])
        a1t = -jnp.exp(a1_ref[j])
        a0t = -jnp.exp(a0_ref[j])
        bb = jnp.broadcast_to(jnp.swapaxes(Bm[:, None, :], 1, 2),
                              (T, D_STATE, D_INNER))
        cb = jnp.broadcast_to(jnp.swapaxes(Cm[:, None, :], 1, 2),
                              (T, D_STATE, D_INNER))
        cb_list.append(cb)
        dAA_ref[:, :, jsl] = jnp.exp(d1[:, None, :] * a1t[None, :, :])
        dBA_ref[:, :, jsl] = (d1 * u0)[:, None, :] * bb
        dAB_ref[:, :, jsl] = jnp.exp(d0[:, None, :] * a0t[None, :, :])
        dBB_ref[:, :, jsl] = (d0 * u1)[:, None, :] * bb

    t0 = jnp.where(cc == 0, 0, T - 1)
    sg = jnp.where(cc == 0, 1, -1)

    def body(t, carry):
        hA, hB = carry
        ts = t0 + sg * t
        hA = dAA_ref[ts] * hA + dBA_ref[ts]
        hB = dAB_ref[ts] * hB + dBB_ref[ts]
        dAA_ref[ts] = hA
        dAB_ref[ts] = hB
        return hA, hB

    hA, hB = lax.fori_loop(0, T, body, (hA_ref[...], hB_ref[...]), unroll=8)
    hA_ref[...] = hA
    hB_ref[...] = hB

    for j in range(2):
        jsl = slice(j * 128, (j + 1) * 128)
        cb = cb_list[j]
        u0, u1 = u_list[j]
        yA = jnp.sum(dAA_ref[:, :, jsl] * cb, axis=1) + u0 * d1_ref[j]
        yB = jnp.sum(dAB_ref[:, :, jsl] * cb, axis=1) + u1 * d0_ref[j]
        y_ref = (yf_ref, yt_ref)[j]
        y_ref[0, 0] = yA
        y_ref[1, 0] = yB


def _scan_call(xf, xt, xpw, dtwc, b0, b1, a0, a1, ds0, ds1):
    T = T_SCAN
    wmap = lambda c, i: (c, 0, 0)
    rowmap = lambda c, i: (0, jnp.where(c == 0, i, NCH - 1 - i), 0)

    def ymap(c, i):
        return (0, c, jnp.where(c == 0, i, NCH - 1 - i), 0)

    return pl.pallas_call(
        _scan_kernel,
        grid=(2, NCH),
        in_specs=[
            pl.BlockSpec((2, T, D_INNER), rowmap),
            pl.BlockSpec((2, T, D_INNER), rowmap),
            pl.BlockSpec((2, DT_RANK + 2 * D_STATE, D_INNER), wmap),
            pl.BlockSpec((2, 2 * D_INNER, DT_RANK), wmap),
            pl.BlockSpec((2, 1, D_INNER), wmap),
            pl.BlockSpec((2, 1, D_INNER), wmap),
            pl.BlockSpec((2, D_STATE, D_INNER), wmap),
            pl.BlockSpec((2, D_STATE, D_INNER), wmap),
            pl.BlockSpec((2, 1, D_INNER), wmap),
            pl.BlockSpec((2, 1, D_INNER), wmap),
        ],
        out_specs=[
            pl.BlockSpec((2, 1, T, D_INNER), ymap),
            pl.BlockSpec((2, 1, T, D_INNER), ymap),
        ],
        out_shape=[
            jax.ShapeDtypeStruct((2, 2, L, D_INNER), jnp.float32),
            jax.ShapeDtypeStruct((2, 2, L, D_INNER), jnp.float32),
        ],
        scratch_shapes=[
            pltpu.VMEM((T, D_STATE, 2 * D_INNER), jnp.float32),
            pltpu.VMEM((T, D_STATE, 2 * D_INNER), jnp.float32),
            pltpu.VMEM((T, D_STATE, 2 * D_INNER), jnp.float32),
            pltpu.VMEM((T, D_STATE, 2 * D_INNER), jnp.float32),
            pltpu.VMEM((D_STATE, 2 * D_INNER), jnp.float32),
            pltpu.VMEM((D_STATE, 2 * D_INNER), jnp.float32),
        ],
        compiler_params=pltpu.CompilerParams(
            dimension_semantics=("arbitrary", "arbitrary"),
            vmem_limit_bytes=56 * 1024 * 1024,
        ),
        name="ss2d_scan",
    )(xf, xt, xpw, dtwc, b0, b1, a0, a1, ds0, ds1)


def _post_kernel(yf_ref, yt_ref, z_ref, g_ref, b_ref, wout_ref, o_ref):
    mf = jnp.sum(yf_ref[0], axis=0)
    yt = jnp.sum(yt_ref[0], axis=0)
    mt = jnp.swapaxes(yt, 0, 1).reshape(T_POST, D_INNER)
    m = mf + mt
    mu = jnp.mean(m, axis=-1, keepdims=True)
    xc = m - mu
    var = jnp.mean(xc * xc, axis=-1, keepdims=True)
    y = xc * lax.rsqrt(var + 1e-5) * g_ref[0] + b_ref[0]
    y = y * _silu(z_ref[0])
    o_ref[0] = _dotT(y, wout_ref[0])


def _post_call(yf, yt4, z, g, b, wout):
    nblk = L // T_POST
    return pl.pallas_call(
        _post_kernel,
        grid=(2, nblk),
        in_specs=[
            pl.BlockSpec((1, 2, T_POST, D_INNER), lambda c, i: (c, 0, i, 0)),
            pl.BlockSpec((1, 2, W, BH, D_INNER), lambda c, i: (c, 0, 0, i, 0)),
            pl.BlockSpec((1, T_POST, D_INNER), lambda c, i: (c, i, 0)),
            pl.BlockSpec((1, 1, D_INNER), lambda c, i: (c, 0, 0)),
            pl.BlockSpec((1, 1, D_INNER), lambda c, i: (c, 0, 0)),
            pl.BlockSpec((1, D_MODEL, D_INNER), lambda c, i: (c, 0, 0)),
        ],
        out_specs=pl.BlockSpec((1, T_POST, D_MODEL), lambda c, i: (c, i, 0)),
        out_shape=jax.ShapeDtypeStruct((2, L, D_MODEL), jnp.float32),
        compiler_params=pltpu.CompilerParams(
            dimension_semantics=("arbitrary", "arbitrary"),
            vmem_limit_bytes=48 * 1024 * 1024,
        ),
        name="ss2d_post",
    )(yf, yt4, z, g, b, wout)


def kernel(x0, x1, in_proj0_w, in_proj1_w, conv0_w, conv0_b, conv1_w, conv1_b,
           x_proj_w0, x_proj_w1, dt_w0, dt_w1, dt_b0, dt_b1,
           A_logs0, A_logs1, Ds0, Ds1, ln0_g, ln0_b, ln1_g, ln1_b,
           out_proj0_w, out_proj1_w):
    xstk = jnp.stack([x0[0], x1[0]])
    win = jnp.stack([in_proj0_w, in_proj1_w])
    cw = jnp.stack([conv0_w.reshape(D_INNER, 9).T,
                    conv1_w.reshape(D_INNER, 9).T])
    cb = jnp.stack([conv0_b, conv1_b]).reshape(2, 1, D_INNER)

    xf, xt, z = _pre_call(xstk, win, cw, cb)

    def a8(al):
        return al.reshape(K, D_INNER, D_STATE).transpose(0, 2, 1)

    b0 = dt_b0.reshape(K, 1, D_INNER)
    b1 = dt_b1.reshape(K, 1, D_INNER)
    ds0 = Ds0.reshape(K, 1, D_INNER)
    ds1 = Ds1.reshape(K, 1, D_INNER)

    dtwc = jnp.concatenate([dt_w1, dt_w0], axis=1)
    yf, yt = _scan_call(xf, xt, x_proj_w0, dtwc, b0, b1,
                        a8(A_logs0), a8(A_logs1), ds0, ds1)

    yt4 = yt.reshape(2, 2, W, H, D_INNER)
    g = jnp.stack([ln0_g, ln1_g]).reshape(2, 1, D_INNER)
    bb = jnp.stack([ln0_b, ln1_b]).reshape(2, 1, D_INNER)
    wout = jnp.stack([out_proj0_w, out_proj1_w])

    out = _post_call(yf, yt4, z, g, bb, wout)
    return (out[0].reshape(1, H, W, D_MODEL),
            out[1].reshape(1, H, W, D_MODEL))


# speedup vs baseline: 1.1063x; 1.1063x over previous
import jax
import jax.numpy as jnp
from jax import lax
from jax.experimental import pallas as pl
from jax.experimental.pallas import tpu as pltpu

D_MODEL = 64
D_STATE = 16
D_INNER = 128
DT_RANK = 4
K = 4
H = 64
W = 64
L = H * W

T_SCAN = 256
NCH = L // T_SCAN
T_POST = 1024
BH = T_POST // W

_HI = jax.lax.Precision.HIGHEST


def _silu(x):
    return x * (1.0 / (1.0 + jnp.exp(-x)))


def _softplus(x):
    return jnp.maximum(x, 0.0) + jnp.log(1.0 + jnp.exp(-jnp.abs(x)))


def _dotT(a, b):
    return lax.dot_general(a, b, (((1,), (1,)), ((), ())),
                           preferred_element_type=jnp.float32)


def _pre_kernel(x_ref, win_ref, cw_ref, cb_ref, xf_ref, xt_ref, z_ref):
    x = x_ref[...].reshape(L, D_MODEL)
    w = win_ref[0]
    xz = _dotT(x, w)
    z_ref[...] = xz[:, D_INNER:].reshape(1, L, D_INNER)
    xi = xz[:, :D_INNER].reshape(H, W, D_INNER)

    cw = cw_ref[0]
    zW = jnp.zeros((H, 1, D_INNER), jnp.float32)
    zH = jnp.zeros((1, W, D_INNER), jnp.float32)
    xm = jnp.concatenate([xi[:, 1:, :], zW], axis=1)
    xp = jnp.concatenate([zW, xi[:, :-1, :]], axis=1)
    taps = []
    for kh in range(3):
        for kw in range(3):
            src_ = (xp, xi, xm)[kw]
            dy = kh - 1
            if dy == -1:
                sh = jnp.concatenate([zH, src_[:-1]], axis=0)
            elif dy == 0:
                sh = src_
            else:
                sh = jnp.concatenate([src_[1:], zH], axis=0)
            taps.append(sh * cw[kh * 3 + kw][None, None, :])
    acc = ((taps[0] + taps[1]) + (taps[2] + taps[3])) + \
          ((taps[4] + taps[5]) + (taps[6] + taps[7])) + taps[8]
    acc = acc + cb_ref[0][None, :, :]
    xc = _silu(acc)
    xf_ref[...] = xc.reshape(1, L, D_INNER)
    xt_ref[...] = jnp.swapaxes(xc, 0, 1).reshape(1, L, D_INNER)


def _pre_call(xstk, win, cw, cb):
    return pl.pallas_call(
        _pre_kernel,
        grid=(2,),
        in_specs=[
            pl.BlockSpec((1, H, W, D_MODEL), lambda c: (c, 0, 0, 0)),
            pl.BlockSpec((1, 2 * D_INNER, D_MODEL), lambda c: (c, 0, 0)),
            pl.BlockSpec((1, 9, D_INNER), lambda c: (c, 0, 0)),
            pl.BlockSpec((1, 1, D_INNER), lambda c: (c, 0, 0)),
        ],
        out_specs=[
            pl.BlockSpec((1, L, D_INNER), lambda c: (c, 0, 0)),
            pl.BlockSpec((1, L, D_INNER), lambda c: (c, 0, 0)),
            pl.BlockSpec((1, L, D_INNER), lambda c: (c, 0, 0)),
        ],
        out_shape=[
            jax.ShapeDtypeStruct((2, L, D_INNER), jnp.float32),
            jax.ShapeDtypeStruct((2, L, D_INNER), jnp.float32),
            jax.ShapeDtypeStruct((2, L, D_INNER), jnp.float32),
        ],
        compiler_params=pltpu.CompilerParams(
            dimension_semantics=("arbitrary",),
            vmem_limit_bytes=48 * 1024 * 1024,
        ),
        name="ss2d_pre",
    )(xstk, win, cw, cb)


def _scan_kernel(xf_ref, xt_ref, xpw_ref, dtw0_ref, dtw1_ref, b0_ref, b1_ref,
                 a0_ref, a1_ref, d0_ref, d1_ref,
                 yf_ref, yt_ref,
                 dAA_ref, dBA_ref, dAB_ref, dBB_ref, hsA_ref, hsB_ref,
                 hA_ref, hB_ref):
    cc = pl.program_id(0)
    i = pl.program_id(1)
    T = T_SCAN

    @pl.when(i == 0)
    def _():
        hA_ref[...] = jnp.zeros_like(hA_ref)
        hB_ref[...] = jnp.zeros_like(hB_ref)

    cb_list = []
    u_list = []
    for j in range(2):
        jsl = slice(j * 128, (j + 1) * 128)
        u_ref = (xf_ref, xt_ref)[j]
        u0 = u_ref[0]
        u1 = u_ref[1]
        u_list.append((u0, u1))
        xd = _dotT(u0, xpw_ref[j])
        r = xd[:, :DT_RANK]
        Bm = xd[:, DT_RANK:DT_RANK + D_STATE]
        Cm = xd[:, DT_RANK + D_STATE:DT_RANK + 2 * D_STATE]
        d1 = _softplus(_dotT(r, dtw1_ref[j]) + b1_ref[j])
        d0 = _softplus(_dotT(r, dtw0_ref[j]) + b0_ref[j])
        a1t = -jnp.exp(a1_ref[j])
        a0t = -jnp.exp(a0_ref[j])
        bb = jnp.broadcast_to(jnp.swapaxes(Bm[:, None, :], 1, 2),
                              (T, D_STATE, D_INNER))
        cb = jnp.broadcast_to(jnp.swapaxes(Cm[:, None, :], 1, 2),
                              (T, D_STATE, D_INNER))
        cb_list.append(cb)
        dAA_ref[:, :, jsl] = jnp.exp(d1[:, None, :] * a1t[None, :, :])
        dBA_ref[:, :, jsl] = (d1 * u0)[:, None, :] * bb
        dAB_ref[:, :, jsl] = jnp.exp(d0[:, None, :] * a0t[None, :, :])
        dBB_ref[:, :, jsl] = (d0 * u1)[:, None, :] * bb

    t0 = jnp.where(cc == 0, 0, T - 1)
    sg = jnp.where(cc == 0, 1, -1)

    def body(t, carry):
        hA, hB = carry
        ts = t0 + sg * t
        hA = dAA_ref[ts] * hA + dBA_ref[ts]
        hB = dAB_ref[ts] * hB + dBB_ref[ts]
        hsA_ref[ts] = hA
        hsB_ref[ts] = hB
        return hA, hB

    hA, hB = lax.fori_loop(0, T, body, (hA_ref[...], hB_ref[...]), unroll=16)
    hA_ref[...] = hA
    hB_ref[...] = hB

    for j in range(2):
        jsl = slice(j * 128, (j + 1) * 128)
        cb = cb_list[j]
        u0, u1 = u_list[j]
        yA = jnp.sum(hsA_ref[:, :, jsl] * cb, axis=1) + u0 * d1_ref[j]
        yB = jnp.sum(hsB_ref[:, :, jsl] * cb, axis=1) + u1 * d0_ref[j]
        y_ref = (yf_ref, yt_ref)[j]
        y_ref[0, 0] = yA
        y_ref[1, 0] = yB


def _scan_call(xf, xt, xpw, dtw0, dtw1, b0, b1, a0, a1, ds0, ds1):
    T = T_SCAN
    wmap = lambda c, i: (c, 0, 0)
    rowmap = lambda c, i: (0, jnp.where(c == 0, i, NCH - 1 - i), 0)

    def ymap(c, i):
        return (0, c, jnp.where(c == 0, i, NCH - 1 - i), 0)

    return pl.pallas_call(
        _scan_kernel,
        grid=(2, NCH),
        in_specs=[
            pl.BlockSpec((2, T, D_INNER), rowmap),
            pl.BlockSpec((2, T, D_INNER), rowmap),
            pl.BlockSpec((2, DT_RANK + 2 * D_STATE, D_INNER), wmap),
            pl.BlockSpec((2, D_INNER, DT_RANK), wmap),
            pl.BlockSpec((2, D_INNER, DT_RANK), wmap),
            pl.BlockSpec((2, 1, D_INNER), wmap),
            pl.BlockSpec((2, 1, D_INNER), wmap),
            pl.BlockSpec((2, D_STATE, D_INNER), wmap),
            pl.BlockSpec((2, D_STATE, D_INNER), wmap),
            pl.BlockSpec((2, 1, D_INNER), wmap),
            pl.BlockSpec((2, 1, D_INNER), wmap),
        ],
        out_specs=[
            pl.BlockSpec((2, 1, T, D_INNER), ymap),
            pl.BlockSpec((2, 1, T, D_INNER), ymap),
        ],
        out_shape=[
            jax.ShapeDtypeStruct((2, 2, L, D_INNER), jnp.float32),
            jax.ShapeDtypeStruct((2, 2, L, D_INNER), jnp.float32),
        ],
        scratch_shapes=[
            pltpu.VMEM((T, D_STATE, 2 * D_INNER), jnp.float32),
            pltpu.VMEM((T, D_STATE, 2 * D_INNER), jnp.float32),
            pltpu.VMEM((T, D_STATE, 2 * D_INNER), jnp.float32),
            pltpu.VMEM((T, D_STATE, 2 * D_INNER), jnp.float32),
            pltpu.VMEM((T, D_STATE, 2 * D_INNER), jnp.float32),
            pltpu.VMEM((T, D_STATE, 2 * D_INNER), jnp.float32),
            pltpu.VMEM((D_STATE, 2 * D_INNER), jnp.float32),
            pltpu.VMEM((D_STATE, 2 * D_INNER), jnp.float32),
        ],
        compiler_params=pltpu.CompilerParams(
            dimension_semantics=("arbitrary", "arbitrary"),
            vmem_limit_bytes=52 * 1024 * 1024,
        ),
        name="ss2d_scan",
    )(xf, xt, xpw, dtw0, dtw1, b0, b1, a0, a1, ds0, ds1)


def _post_kernel(yf_ref, yt_ref, z_ref, g_ref, b_ref, wout_ref, o_ref):
    mf = jnp.sum(yf_ref[0], axis=0)
    yt = jnp.sum(yt_ref[0], axis=0)
    mt = jnp.swapaxes(yt, 0, 1).reshape(T_POST, D_INNER)
    m = mf + mt
    mu = jnp.mean(m, axis=-1, keepdims=True)
    xc = m - mu
    var = jnp.mean(xc * xc, axis=-1, keepdims=True)
    y = xc * lax.rsqrt(var + 1e-5) * g_ref[0] + b_ref[0]
    y = y * _silu(z_ref[0])
    o_ref[0] = _dotT(y, wout_ref[0])


def _post_call(yf, yt4, z, g, b, wout):
    nblk = L // T_POST
    return pl.pallas_call(
        _post_kernel,
        grid=(2, nblk),
        in_specs=[
            pl.BlockSpec((1, 2, T_POST, D_INNER), lambda c, i: (c, 0, i, 0)),
            pl.BlockSpec((1, 2, W, BH, D_INNER), lambda c, i: (c, 0, 0, i, 0)),
            pl.BlockSpec((1, T_POST, D_INNER), lambda c, i: (c, i, 0)),
            pl.BlockSpec((1, 1, D_INNER), lambda c, i: (c, 0, 0)),
            pl.BlockSpec((1, 1, D_INNER), lambda c, i: (c, 0, 0)),
            pl.BlockSpec((1, D_MODEL, D_INNER), lambda c, i: (c, 0, 0)),
        ],
        out_specs=pl.BlockSpec((1, T_POST, D_MODEL), lambda c, i: (c, i, 0)),
        out_shape=jax.ShapeDtypeStruct((2, L, D_MODEL), jnp.float32),
        compiler_params=pltpu.CompilerParams(
            dimension_semantics=("arbitrary", "arbitrary"),
            vmem_limit_bytes=48 * 1024 * 1024,
        ),
        name="ss2d_post",
    )(yf, yt4, z, g, b, wout)


def kernel(x0, x1, in_proj0_w, in_proj1_w, conv0_w, conv0_b, conv1_w, conv1_b,
           x_proj_w0, x_proj_w1, dt_w0, dt_w1, dt_b0, dt_b1,
           A_logs0, A_logs1, Ds0, Ds1, ln0_g, ln0_b, ln1_g, ln1_b,
           out_proj0_w, out_proj1_w):
    xstk = jnp.stack([x0[0], x1[0]])
    win = jnp.stack([in_proj0_w, in_proj1_w])
    cw = jnp.stack([conv0_w.reshape(D_INNER, 9).T,
                    conv1_w.reshape(D_INNER, 9).T])
    cb = jnp.stack([conv0_b, conv1_b]).reshape(2, 1, D_INNER)

    xf, xt, z = _pre_call(xstk, win, cw, cb)

    def a8(al):
        return al.reshape(K, D_INNER, D_STATE).transpose(0, 2, 1)

    b0 = dt_b0.reshape(K, 1, D_INNER)
    b1 = dt_b1.reshape(K, 1, D_INNER)
    ds0 = Ds0.reshape(K, 1, D_INNER)
    ds1 = Ds1.reshape(K, 1, D_INNER)

    yf, yt = _scan_call(xf, xt, x_proj_w0, dt_w0, dt_w1, b0, b1,
                        a8(A_logs0), a8(A_logs1), ds0, ds1)

    yt4 = yt.reshape(2, 2, W, H, D_INNER)
    g = jnp.stack([ln0_g, ln1_g]).reshape(2, 1, D_INNER)
    bb = jnp.stack([ln0_b, ln1_b]).reshape(2, 1, D_INNER)
    wout = jnp.stack([out_proj0_w, out_proj1_w])

    out = _post_call(yf, yt4, z, g, bb, wout)
    return (out[0].reshape(1, H, W, D_MODEL),
            out[1].reshape(1, H, W, D_MODEL))


# speedup vs baseline: 1.5364x; 1.3887x over previous
import jax
import jax.numpy as jnp
from jax import lax
from jax.experimental import pallas as pl
from jax.experimental.pallas import tpu as pltpu

D_MODEL = 64
D_STATE = 16
D_INNER = 128
DT_RANK = 4
K = 4
H = 64
W = 64
L = H * W

T_SCAN = 256
NCH = L // T_SCAN
T_POST = 1024
BH = T_POST // W

_HI = jax.lax.Precision.HIGHEST


def _silu(x):
    return x * (1.0 / (1.0 + jnp.exp(-x)))


def _softplus(x):
    return jnp.maximum(x, 0.0) + jnp.log(1.0 + jnp.exp(-jnp.abs(x)))


def _dotT(a, b):
    return lax.dot_general(a, b, (((1,), (1,)), ((), ())),
                           preferred_element_type=jnp.float32)


def _pre_kernel(x_ref, win_ref, cw_ref, cb_ref, xf_ref, xt_ref, z_ref):
    x = x_ref[...].reshape(L, D_MODEL)
    w = win_ref[0]
    xz = _dotT(x, w)
    z_ref[...] = xz[:, D_INNER:].reshape(1, L, D_INNER)
    xi = xz[:, :D_INNER].reshape(H, W, D_INNER)

    cw = cw_ref[0]
    zW = jnp.zeros((H, 1, D_INNER), jnp.float32)
    zH = jnp.zeros((1, W, D_INNER), jnp.float32)
    xm = jnp.concatenate([xi[:, 1:, :], zW], axis=1)
    xp = jnp.concatenate([zW, xi[:, :-1, :]], axis=1)
    taps = []
    for kh in range(3):
        for kw in range(3):
            src_ = (xp, xi, xm)[kw]
            dy = kh - 1
            if dy == -1:
                sh = jnp.concatenate([zH, src_[:-1]], axis=0)
            elif dy == 0:
                sh = src_
            else:
                sh = jnp.concatenate([src_[1:], zH], axis=0)
            taps.append(sh * cw[kh * 3 + kw][None, None, :])
    acc = ((taps[0] + taps[1]) + (taps[2] + taps[3])) + \
          ((taps[4] + taps[5]) + (taps[6] + taps[7])) + taps[8]
    acc = acc + cb_ref[0][None, :, :]
    xc = _silu(acc)
    xf_ref[...] = xc.reshape(1, L, D_INNER)
    xt_ref[...] = jnp.swapaxes(xc, 0, 1).reshape(1, L, D_INNER)


def _pre_call(xstk, win, cw, cb):
    return pl.pallas_call(
        _pre_kernel,
        grid=(2,),
        in_specs=[
            pl.BlockSpec((1, H, W, D_MODEL), lambda c: (c, 0, 0, 0)),
            pl.BlockSpec((1, 2 * D_INNER, D_MODEL), lambda c: (c, 0, 0)),
            pl.BlockSpec((1, 9, D_INNER), lambda c: (c, 0, 0)),
            pl.BlockSpec((1, 1, D_INNER), lambda c: (c, 0, 0)),
        ],
        out_specs=[
            pl.BlockSpec((1, L, D_INNER), lambda c: (c, 0, 0)),
            pl.BlockSpec((1, L, D_INNER), lambda c: (c, 0, 0)),
            pl.BlockSpec((1, L, D_INNER), lambda c: (c, 0, 0)),
        ],
        out_shape=[
            jax.ShapeDtypeStruct((2, L, D_INNER), jnp.float32),
            jax.ShapeDtypeStruct((2, L, D_INNER), jnp.float32),
            jax.ShapeDtypeStruct((2, L, D_INNER), jnp.float32),
        ],
        compiler_params=pltpu.CompilerParams(
            dimension_semantics=("arbitrary",),
            vmem_limit_bytes=48 * 1024 * 1024,
        ),
        name="ss2d_pre",
    )(xstk, win, cw, cb)


def _scan_kernel(xf_ref, xt_ref, xpw_ref, dtw0_ref, dtw1_ref, b0_ref, b1_ref,
                 a0_ref, a1_ref, d0_ref, d1_ref,
                 yf_ref, yt_ref,
                 dAA_ref, dBA_ref, dAB_ref, dBB_ref, hsA_ref, hsB_ref,
                 hA_ref, hB_ref):
    cc = pl.program_id(0)
    i = pl.program_id(1)
    T = T_SCAN

    @pl.when(i == 0)
    def _():
        hA_ref[...] = jnp.zeros_like(hA_ref)
        hB_ref[...] = jnp.zeros_like(hB_ref)

    cb_list = []
    u_list = []
    for j in range(2):
        jsl = slice(j * 128, (j + 1) * 128)
        u_ref = (xf_ref, xt_ref)[j]
        u0 = u_ref[0]
        u1 = u_ref[1]
        u_list.append((u0, u1))
        xd = _dotT(u0, xpw_ref[j])
        r = xd[:, :DT_RANK]
        Bm = xd[:, DT_RANK:DT_RANK + D_STATE]
        Cm = xd[:, DT_RANK + D_STATE:DT_RANK + 2 * D_STATE]
        d1 = _softplus(_dotT(r, dtw1_ref[j]) + b1_ref[j])
        d0 = _softplus(_dotT(r, dtw0_ref[j]) + b0_ref[j])
        a1t = -jnp.exp(a1_ref[j])
        a0t = -jnp.exp(a0_ref[j])
        xd_rep = jnp.broadcast_to(xd[:, None, :], (T, D_STATE, 36))
        xd_rep = xd_rep.reshape(T * D_STATE, 36)
        r16 = lax.broadcasted_iota(jnp.int32, (T * D_STATE, 36), 0) & 15
        cl = lax.broadcasted_iota(jnp.int32, (T * D_STATE, 36), 1)
        selB = jnp.where(cl == r16 + DT_RANK, xd_rep, 0.0)
        selC = jnp.where(cl == r16 + DT_RANK + D_STATE, xd_rep, 0.0)
        sel = jnp.concatenate([selB, selC], axis=1)
        mrow = lax.broadcasted_iota(jnp.int32, (72, 2 * D_INNER), 0)
        mcol = lax.broadcasted_iota(jnp.int32, (72, 2 * D_INNER), 1)
        mbc = jnp.where((mrow < 36) == (mcol < D_INNER), 1.0, 0.0)
        out2 = lax.dot_general(sel, mbc, (((1,), (0,)), ((), ())),
                               preferred_element_type=jnp.float32)
        bb = out2[:, :D_INNER].reshape(T, D_STATE, D_INNER)
        cb = out2[:, D_INNER:].reshape(T, D_STATE, D_INNER)
        cb_list.append(cb)
        dAA_ref[:, :, jsl] = jnp.exp(d1[:, None, :] * a1t[None, :, :])
        dBA_ref[:, :, jsl] = (d1 * u0)[:, None, :] * bb
        dAB_ref[:, :, jsl] = jnp.exp(d0[:, None, :] * a0t[None, :, :])
        dBB_ref[:, :, jsl] = (d0 * u1)[:, None, :] * bb

    t0 = jnp.where(cc == 0, 0, T - 1)
    sg = jnp.where(cc == 0, 1, -1)

    def body(t, carry):
        hA, hB = carry
        ts = t0 + sg * t
        hA = dAA_ref[ts] * hA + dBA_ref[ts]
        hB = dAB_ref[ts] * hB + dBB_ref[ts]
        hsA_ref[ts] = hA
        hsB_ref[ts] = hB
        return hA, hB

    hA, hB = lax.fori_loop(0, T, body, (hA_ref[...], hB_ref[...]), unroll=16)
    hA_ref[...] = hA
    hB_ref[...] = hB

    for j in range(2):
        jsl = slice(j * 128, (j + 1) * 128)
        cb = cb_list[j]
        u0, u1 = u_list[j]
        yA = jnp.sum(hsA_ref[:, :, jsl] * cb, axis=1) + u0 * d1_ref[j]
        yB = jnp.sum(hsB_ref[:, :, jsl] * cb, axis=1) + u1 * d0_ref[j]
        y_ref = (yf_ref, yt_ref)[j]
        y_ref[0, 0] = yA
        y_ref[1, 0] = yB


def _scan_call(xf, xt, xpw, dtw0, dtw1, b0, b1, a0, a1, ds0, ds1):
    T = T_SCAN
    wmap = lambda c, i: (c, 0, 0)
    rowmap = lambda c, i: (0, jnp.where(c == 0, i, NCH - 1 - i), 0)

    def ymap(c, i):
        return (0, c, jnp.where(c == 0, i, NCH - 1 - i), 0)

    return pl.pallas_call(
        _scan_kernel,
        grid=(2, NCH),
        in_specs=[
            pl.BlockSpec((2, T, D_INNER), rowmap),
            pl.BlockSpec((2, T, D_INNER), rowmap),
            pl.BlockSpec((2, DT_RANK + 2 * D_STATE, D_INNER), wmap),
            pl.BlockSpec((2, D_INNER, DT_RANK), wmap),
            pl.BlockSpec((2, D_INNER, DT_RANK), wmap),
            pl.BlockSpec((2, 1, D_INNER), wmap),
            pl.BlockSpec((2, 1, D_INNER), wmap),
            pl.BlockSpec((2, D_STATE, D_INNER), wmap),
            pl.BlockSpec((2, D_STATE, D_INNER), wmap),
            pl.BlockSpec((2, 1, D_INNER), wmap),
            pl.BlockSpec((2, 1, D_INNER), wmap),
        ],
        out_specs=[
            pl.BlockSpec((2, 1, T, D_INNER), ymap),
            pl.BlockSpec((2, 1, T, D_INNER), ymap),
        ],
        out_shape=[
            jax.ShapeDtypeStruct((2, 2, L, D_INNER), jnp.float32),
            jax.ShapeDtypeStruct((2, 2, L, D_INNER), jnp.float32),
        ],
        scratch_shapes=[
            pltpu.VMEM((T, D_STATE, 2 * D_INNER), jnp.float32),
            pltpu.VMEM((T, D_STATE, 2 * D_INNER), jnp.float32),
            pltpu.VMEM((T, D_STATE, 2 * D_INNER), jnp.float32),
            pltpu.VMEM((T, D_STATE, 2 * D_INNER), jnp.float32),
            pltpu.VMEM((T, D_STATE, 2 * D_INNER), jnp.float32),
            pltpu.VMEM((T, D_STATE, 2 * D_INNER), jnp.float32),
            pltpu.VMEM((D_STATE, 2 * D_INNER), jnp.float32),
            pltpu.VMEM((D_STATE, 2 * D_INNER), jnp.float32),
        ],
        compiler_params=pltpu.CompilerParams(
            dimension_semantics=("arbitrary", "arbitrary"),
            vmem_limit_bytes=52 * 1024 * 1024,
        ),
        name="ss2d_scan",
    )(xf, xt, xpw, dtw0, dtw1, b0, b1, a0, a1, ds0, ds1)


def _post_kernel(yf_ref, yt_ref, z_ref, g_ref, b_ref, wout_ref, o_ref):
    mf = jnp.sum(yf_ref[0], axis=0)
    yt = jnp.sum(yt_ref[0], axis=0)
    mt = jnp.swapaxes(yt, 0, 1).reshape(T_POST, D_INNER)
    m = mf + mt
    mu = jnp.mean(m, axis=-1, keepdims=True)
    xc = m - mu
    var = jnp.mean(xc * xc, axis=-1, keepdims=True)
    y = xc * lax.rsqrt(var + 1e-5) * g_ref[0] + b_ref[0]
    y = y * _silu(z_ref[0])
    o_ref[0] = _dotT(y, wout_ref[0])


def _post_call(yf, yt4, z, g, b, wout):
    nblk = L // T_POST
    return pl.pallas_call(
        _post_kernel,
        grid=(2, nblk),
        in_specs=[
            pl.BlockSpec((1, 2, T_POST, D_INNER), lambda c, i: (c, 0, i, 0)),
            pl.BlockSpec((1, 2, W, BH, D_INNER), lambda c, i: (c, 0, 0, i, 0)),
            pl.BlockSpec((1, T_POST, D_INNER), lambda c, i: (c, i, 0)),
            pl.BlockSpec((1, 1, D_INNER), lambda c, i: (c, 0, 0)),
            pl.BlockSpec((1, 1, D_INNER), lambda c, i: (c, 0, 0)),
            pl.BlockSpec((1, D_MODEL, D_INNER), lambda c, i: (c, 0, 0)),
        ],
        out_specs=pl.BlockSpec((1, T_POST, D_MODEL), lambda c, i: (c, i, 0)),
        out_shape=jax.ShapeDtypeStruct((2, L, D_MODEL), jnp.float32),
        compiler_params=pltpu.CompilerParams(
            dimension_semantics=("arbitrary", "arbitrary"),
            vmem_limit_bytes=48 * 1024 * 1024,
        ),
        name="ss2d_post",
    )(yf, yt4, z, g, b, wout)


def kernel(x0, x1, in_proj0_w, in_proj1_w, conv0_w, conv0_b, conv1_w, conv1_b,
           x_proj_w0, x_proj_w1, dt_w0, dt_w1, dt_b0, dt_b1,
           A_logs0, A_logs1, Ds0, Ds1, ln0_g, ln0_b, ln1_g, ln1_b,
           out_proj0_w, out_proj1_w):
    xstk = jnp.stack([x0[0], x1[0]])
    win = jnp.stack([in_proj0_w, in_proj1_w])
    cw = jnp.stack([conv0_w.reshape(D_INNER, 9).T,
                    conv1_w.reshape(D_INNER, 9).T])
    cb = jnp.stack([conv0_b, conv1_b]).reshape(2, 1, D_INNER)

    xf, xt, z = _pre_call(xstk, win, cw, cb)

    def a8(al):
        return al.reshape(K, D_INNER, D_STATE).transpose(0, 2, 1)

    b0 = dt_b0.reshape(K, 1, D_INNER)
    b1 = dt_b1.reshape(K, 1, D_INNER)
    ds0 = Ds0.reshape(K, 1, D_INNER)
    ds1 = Ds1.reshape(K, 1, D_INNER)

    yf, yt = _scan_call(xf, xt, x_proj_w0, dt_w0, dt_w1, b0, b1,
                        a8(A_logs0), a8(A_logs1), ds0, ds1)

    yt4 = yt.reshape(2, 2, W, H, D_INNER)
    g = jnp.stack([ln0_g, ln1_g]).reshape(2, 1, D_INNER)
    bb = jnp.stack([ln0_b, ln1_b]).reshape(2, 1, D_INNER)
    wout = jnp.stack([out_proj0_w, out_proj1_w])

    out = _post_call(yf, yt4, z, g, bb, wout)
    return (out[0].reshape(1, H, W, D_MODEL),
            out[1].reshape(1, H, W, D_MODEL))
